# Initial kernel scaffold; baseline (speedup 1.0000x reference)
import jax
import jax.numpy as jnp
from jax.experimental import pallas as pl


def kernel(feats, edge_index, edge_weight, eps, w1, b1, w2, b2, wmu, bmu, wlv, blv, log_alpha, dec_bias, fw1, fb1, fw2, fb2, fw3, fb3):
    raise NotImplementedError("write your pallas kernel here")


# speedup vs baseline: 6.9116x; 6.9116x over previous
import functools

import jax
import jax.numpy as jnp
from jax import lax
from jax.experimental import pallas as pl
from jax.experimental.pallas import tpu as pltpu
from jax.experimental.pallas import tpu_sc as plsc

N = 10000
E = 320000
D = 128
H = 64
L = 16

SUB = 128
NSUB = E // SUB
NW = 32
NPAD = 10240
ROWS_PER_TILE = NPAD // 16
HK = H // 16

_mesh = plsc.VectorSubcoreMesh(core_axis_name="c", subcore_axis_name="s")


def _seg_sum_body(hw_hbm, sd_hbm, w_hbm, out_hbm, ebuf, wbuf, rows, acc, sem):
    c = lax.axis_index("c")
    s = lax.axis_index("s")
    wid = s * 2 + c

    z16 = jnp.zeros((16,), jnp.float32)

    def zero_body(i, _):
        for k in range(HK):
            rows[i, pl.ds(k * 16, 16)] = z16
        return 0

    lax.fori_loop(0, SUB, zero_body, 0)
    for j in range(ROWS_PER_TILE // SUB):
        pltpu.sync_copy(rows, acc.at[pl.ds(s * ROWS_PER_TILE + j * SUB, SUB)])
    plsc.subcore_barrier()

    nq = 78 + jnp.where(wid < NSUB - 32 * 78, 1, 0)

    def block_body(t, _):
        q = wid + 32 * t
        pltpu.sync_copy(sd_hbm.at[q], ebuf)
        pltpu.sync_copy(w_hbm.at[q], wbuf)
        pltpu.async_copy(hw_hbm.at[ebuf.at[0]], rows, sem).wait()

        for g in range(SUB // 16):
            w16 = wbuf[pl.ds(g * 16, 16)]
            for i in range(16):
                wb = w16.at[jnp.full((16,), i, jnp.int32)].get(
                    mode="promise_in_bounds")
                e = g * 16 + i
                for k in range(HK):
                    sl = pl.ds(k * 16, 16)
                    rows[e, sl] = rows[e, sl] * wb
        pltpu.sync_copy(rows, acc.at[ebuf.at[1]], add=True)
        return 0

    lax.fori_loop(0, nq, block_body, 0)
    plsc.subcore_barrier()
    pltpu.sync_copy(acc.at[pl.ds(s * ROWS_PER_TILE, ROWS_PER_TILE)],
                    out_hbm.at[c, pl.ds(s * ROWS_PER_TILE, ROWS_PER_TILE)])


_seg_sum = pl.kernel(
    _seg_sum_body,
    out_type=jax.ShapeDtypeStruct((2, NPAD, H), jnp.float32),
    mesh=_mesh,
    compiler_params=pltpu.CompilerParams(use_tc_tiling_on_sc=False),
    scratch_types=[
        pltpu.VMEM((2, SUB), jnp.int32),
        pltpu.VMEM((SUB,), jnp.float32),
        pltpu.VMEM((SUB, H), jnp.float32),
        pltpu.VMEM_SHARED((NPAD, H), jnp.float32),
        pltpu.SemaphoreType.DMA,
    ],
)


def _zgather_body(z_hbm, sd_hbm, zi_hbm, zj_hbm, ebuf, zrows, sem):
    c = lax.axis_index("c")
    s = lax.axis_index("s")
    wid = s * 2 + c
    nq = 78 + jnp.where(wid < NSUB - 32 * 78, 1, 0)

    def block_body(t, _):
        q = wid + 32 * t
        pltpu.sync_copy(sd_hbm.at[q], ebuf)
        pltpu.async_copy(z_hbm.at[ebuf.at[0]], zrows, sem).wait()
        pltpu.sync_copy(zrows, zi_hbm.at[pl.ds(q * SUB, SUB)])
        pltpu.async_copy(z_hbm.at[ebuf.at[1]], zrows, sem).wait()
        pltpu.sync_copy(zrows, zj_hbm.at[pl.ds(q * SUB, SUB)])
        return 0

    lax.fori_loop(0, nq, block_body, 0)


_zgather = pl.kernel(
    _zgather_body,
    out_type=(jax.ShapeDtypeStruct((E, L), jnp.float32),
              jax.ShapeDtypeStruct((E, L), jnp.float32)),
    mesh=_mesh,
    compiler_params=pltpu.CompilerParams(use_tc_tiling_on_sc=False),
    scratch_types=[
        pltpu.VMEM((2, SUB), jnp.int32),
        pltpu.VMEM((SUB, L), jnp.float32),
        pltpu.SemaphoreType.DMA,
    ],
)


BM = 1000


def _mm_bias_kernel(x_ref, w_ref, b_ref, o_ref):
    o_ref[...] = (jnp.dot(x_ref[...], w_ref[...],
                          preferred_element_type=jnp.float32) + b_ref[...])


def _mm_bias(x, w, b):
    m, d = x.shape
    h = w.shape[1]
    return pl.pallas_call(
        _mm_bias_kernel,
        grid=(m // BM,),
        in_specs=[
            pl.BlockSpec((BM, d), lambda i: (i, 0)),
            pl.BlockSpec((d, h), lambda i: (0, 0)),
            pl.BlockSpec((1, h), lambda i: (0, 0)),
        ],
        out_specs=pl.BlockSpec((BM, h), lambda i: (i, 0)),
        out_shape=jax.ShapeDtypeStruct((m, h), jnp.float32),
    )(x, w, b.reshape(1, h))


def _relu_mm_kernel(p_ref, w_ref, b_ref, o_ref):
    h = jax.nn.relu(p_ref[0] + p_ref[1])
    o_ref[...] = (jnp.dot(h, w_ref[...],
                          preferred_element_type=jnp.float32) + b_ref[...])


def _relu_mm(p, w, b):
    h = w.shape[1]
    return pl.pallas_call(
        _relu_mm_kernel,
        grid=(N // BM,),
        in_specs=[
            pl.BlockSpec((2, BM, H), lambda i: (0, i, 0)),
            pl.BlockSpec((H, h), lambda i: (0, 0)),
            pl.BlockSpec((1, h), lambda i: (0, 0)),
        ],
        out_specs=pl.BlockSpec((BM, h), lambda i: (i, 0)),
        out_shape=jax.ShapeDtypeStruct((N, h), jnp.float32),
    )(p, w, b.reshape(1, h))


def _heads_kernel(p_ref, eps_ref, wmu_ref, bmu_ref, wlv_ref, blv_ref,
                  fw1_ref, fb1_ref, fw2_ref, fb2_ref, fw3_ref, fb3_ref,
                  mu_ref, lv_ref, z_ref, xr_ref):
    h2 = jax.nn.relu(p_ref[0] + p_ref[1])
    mu = jnp.dot(h2, wmu_ref[...], preferred_element_type=jnp.float32) + bmu_ref[...]
    lv = jnp.dot(h2, wlv_ref[...], preferred_element_type=jnp.float32) + blv_ref[...]
    z = mu + jnp.exp(0.5 * lv) * eps_ref[...]
    mu_ref[...] = mu
    lv_ref[...] = lv
    z_ref[...] = z
    hx = jax.nn.relu(jnp.dot(z, fw1_ref[...], preferred_element_type=jnp.float32)
                     + fb1_ref[...])
    hx = jax.nn.relu(jnp.dot(hx, fw2_ref[...], preferred_element_type=jnp.float32)
                     + fb2_ref[...])
    xr_ref[...] = (jnp.dot(hx, fw3_ref[...], preferred_element_type=jnp.float32)
                   + fb3_ref[...])


def _heads(p, eps, wmu, bmu, wlv, blv, fw1, fb1, fw2, fb2, fw3, fb3):
    f1 = fw1.shape[1]
    return pl.pallas_call(
        _heads_kernel,
        grid=(N // BM,),
        in_specs=[
            pl.BlockSpec((2, BM, H), lambda i: (0, i, 0)),
            pl.BlockSpec((BM, L), lambda i: (i, 0)),
            pl.BlockSpec((H, L), lambda i: (0, 0)),
            pl.BlockSpec((1, L), lambda i: (0, 0)),
            pl.BlockSpec((H, L), lambda i: (0, 0)),
            pl.BlockSpec((1, L), lambda i: (0, 0)),
            pl.BlockSpec((L, f1), lambda i: (0, 0)),
            pl.BlockSpec((1, f1), lambda i: (0, 0)),
            pl.BlockSpec((f1, f1), lambda i: (0, 0)),
            pl.BlockSpec((1, f1), lambda i: (0, 0)),
            pl.BlockSpec((f1, D), lambda i: (0, 0)),
            pl.BlockSpec((1, D), lambda i: (0, 0)),
        ],
        out_specs=[
            pl.BlockSpec((BM, L), lambda i: (i, 0)),
            pl.BlockSpec((BM, L), lambda i: (i, 0)),
            pl.BlockSpec((BM, L), lambda i: (i, 0)),
            pl.BlockSpec((BM, D), lambda i: (i, 0)),
        ],
        out_shape=[
            jax.ShapeDtypeStruct((N, L), jnp.float32),
            jax.ShapeDtypeStruct((N, L), jnp.float32),
            jax.ShapeDtypeStruct((N, L), jnp.float32),
            jax.ShapeDtypeStruct((N, D), jnp.float32),
        ],
    )(p, eps, wmu, bmu.reshape(1, L), wlv, blv.reshape(1, L),
      fw1, fb1.reshape(1, f1), fw2, fb2.reshape(1, f1), fw3, fb3.reshape(1, D))


EBM = 4000


def _edge_logits_kernel(zi_ref, zj_ref, s_ref, la_ref, db_ref, o_ref):
    d = zi_ref[...] - zj_ref[...]
    dist2 = jnp.dot(d * d, s_ref[...], preferred_element_type=jnp.float32)
    la = la_ref[0, 0]
    alpha = jnp.maximum(la, 0.0) + jnp.log1p(jnp.exp(-jnp.abs(la))) + 0.0001
    o_ref[...] = db_ref[0, 0] - alpha * dist2


def _edge_logits(zi8, zj8, smat, log_alpha, dec_bias):
    g = E // 8
    return pl.pallas_call(
        _edge_logits_kernel,
        grid=(g // EBM,),
        in_specs=[
            pl.BlockSpec((EBM, 128), lambda i: (i, 0)),
            pl.BlockSpec((EBM, 128), lambda i: (i, 0)),
            pl.BlockSpec((128, 8), lambda i: (0, 0)),
            pl.BlockSpec((1, 1), lambda i: (0, 0), memory_space=pltpu.SMEM),
            pl.BlockSpec((1, 1), lambda i: (0, 0), memory_space=pltpu.SMEM),
        ],
        out_specs=pl.BlockSpec((EBM, 8), lambda i: (i, 0)),
        out_shape=jax.ShapeDtypeStruct((g, 8), jnp.float32),
    )(zi8, zj8, smat, log_alpha.reshape(1, 1), dec_bias.reshape(1, 1))


def kernel(feats, edge_index, edge_weight, eps, w1, b1, w2, b2, wmu, bmu,
           wlv, blv, log_alpha, dec_bias, fw1, fb1, fw2, fb2, fw3, fb3):
    ei = edge_index.astype(jnp.int32)
    sd = jnp.stack([ei[0].reshape(NSUB, SUB), ei[1].reshape(NSUB, SUB)], axis=1)
    wsub = edge_weight.reshape(NSUB, SUB)

    hw1 = _mm_bias(feats, w1, b1)
    p1 = _seg_sum(hw1, sd, wsub)
    hw2 = _relu_mm(p1, w2, b2)
    p2 = _seg_sum(hw2, sd, wsub)
    mu, logvar, z, x_recon = _heads(
        p2, eps, wmu, bmu, wlv, blv,
        fw1, fb1, fw2, fb2, fw3, fb3)
    zi, zj = _zgather(z, sd)
    smat = jnp.kron(jnp.eye(8, dtype=jnp.float32),
                    jnp.ones((16, 1), dtype=jnp.float32))
    logits8 = _edge_logits(zi.reshape(E // 8, 128), zj.reshape(E // 8, 128),
                           smat, log_alpha, dec_bias)
    edge_logits = logits8.reshape(E)
    return (edge_logits, x_recon, mu, logvar)
